# Initial kernel scaffold; baseline (speedup 1.0000x reference)
import jax
import jax.numpy as jnp
from jax.experimental import pallas as pl


def kernel(x, lookup_table):
    raise NotImplementedError("write your pallas kernel here")


# speedup vs baseline: 1.2793x; 1.2793x over previous
import functools

import jax
import jax.numpy as jnp
from jax import lax
from jax.experimental import pallas as pl
from jax.experimental.pallas import tpu as pltpu
from jax.experimental.pallas import tpu_sc as plsc

VOCAB = 256
BATCH = 16384
D = 256

_info = plsc.get_sparse_core_info()
_NC, _NS, _L = _info.num_cores, _info.num_subcores, _info.num_lanes
_NW = _NC * _NS
_BPW = BATCH // _NW
_C = 128
_NCHUNK = _BPW // _C


@functools.partial(
    pl.kernel,
    mesh=plsc.VectorSubcoreMesh(core_axis_name="c", subcore_axis_name="s"),
    out_type=jax.ShapeDtypeStruct((BATCH, D), jnp.float32),
    scratch_types=[
        pltpu.VMEM((_BPW,), jnp.int32),
        pltpu.VMEM((_BPW,), jnp.int32),
        pltpu.VMEM((_NCHUNK, _C), jnp.int32),
        pltpu.VMEM((_C, D), jnp.float32),
        pltpu.SemaphoreType.DMA,
    ],
)
def _lookup(a_hbm, b_hbm, table_hbm, out_hbm, a_v, b_v, idx_v, rows_v, gsem):
    wid = lax.axis_index("s") * _NC + lax.axis_index("c")
    base = wid * _BPW
    pltpu.sync_copy(a_hbm.at[pl.ds(base, _BPW)], a_v)
    pltpu.sync_copy(b_hbm.at[pl.ds(base, _BPW)], b_v)
    for i in range(_BPW // _L):
        va = a_v[pl.ds(i * _L, _L)]
        vb = b_v[pl.ds(i * _L, _L)]
        idx_v[i // (_C // _L), pl.ds((i % (_C // _L)) * _L, _L)] = va * VOCAB + vb
    for c in range(_NCHUNK):
        pltpu.async_copy(table_hbm.at[idx_v.at[c]], rows_v, gsem).wait()
        pltpu.sync_copy(rows_v, out_hbm.at[pl.ds(base + c * _C, _C)])


def kernel(x, lookup_table):
    return _lookup(x[:, 0], x[:, 1], lookup_table)


# speedup vs baseline: 1.3535x; 1.0580x over previous
import functools

import jax
import jax.numpy as jnp
from jax import lax
from jax.experimental import pallas as pl
from jax.experimental.pallas import tpu as pltpu
from jax.experimental.pallas import tpu_sc as plsc

VOCAB = 256
BATCH = 16384
D = 256

_info = plsc.get_sparse_core_info()
_NC, _NS, _L = _info.num_cores, _info.num_subcores, _info.num_lanes
_NW = _NC * _NS
_BPW = BATCH // _NW
_C = 128
_NCHUNK = _BPW // _C


@functools.partial(
    pl.kernel,
    mesh=plsc.VectorSubcoreMesh(core_axis_name="c", subcore_axis_name="s"),
    out_type=jax.ShapeDtypeStruct((BATCH, D), jnp.float32),
    scratch_types=[
        pltpu.VMEM((_BPW,), jnp.int32),
        pltpu.VMEM((_BPW,), jnp.int32),
        pltpu.VMEM((_NCHUNK, _C), jnp.int32),
        pltpu.VMEM((3, _C, D), jnp.float32),
        pltpu.SemaphoreType.DMA,
        pltpu.SemaphoreType.DMA,
        pltpu.SemaphoreType.DMA,
        pltpu.SemaphoreType.DMA,
        pltpu.SemaphoreType.DMA,
        pltpu.SemaphoreType.DMA,
    ],
)
def _lookup(a_hbm, b_hbm, table_hbm, out_hbm, a_v, b_v, idx_v, rows_v,
            g0, g1, g2, o0, o1, o2):
    wid = lax.axis_index("s") * _NC + lax.axis_index("c")
    base = wid * _BPW
    gsems = (g0, g1, g2)
    osems = (o0, o1, o2)
    pltpu.sync_copy(a_hbm.at[pl.ds(base, _BPW)], a_v)
    pltpu.sync_copy(b_hbm.at[pl.ds(base, _BPW)], b_v)

    def compute_idx(c):
        for i in range(_C // _L):
            j = c * (_C // _L) + i
            va = a_v[pl.ds(j * _L, _L)]
            vb = b_v[pl.ds(j * _L, _L)]
            idx_v[c, pl.ds(i * _L, _L)] = va * VOCAB + vb

    def gather(c):
        return pltpu.async_copy(table_hbm.at[idx_v.at[c]], rows_v.at[c % 3],
                                gsems[c % 3])

    g = [None] * _NCHUNK
    o = [None] * _NCHUNK
    compute_idx(0)
    g[0] = gather(0)
    compute_idx(1)
    g[1] = gather(1)
    for c in range(2, _NCHUNK):
        compute_idx(c)
    for c in range(_NCHUNK):
        g[c].wait()
        o[c] = pltpu.async_copy(rows_v.at[c % 3],
                                out_hbm.at[pl.ds(base + c * _C, _C)],
                                osems[c % 3])
        if c + 2 < _NCHUNK:
            if c >= 1:
                o[c - 1].wait()
            g[c + 2] = gather(c + 2)
    o[_NCHUNK - 2].wait()
    o[_NCHUNK - 1].wait()


def kernel(x, lookup_table):
    return _lookup(x[:, 0], x[:, 1], lookup_table)
